# baseline (speedup 1.0000x reference)
import functools

import jax
import jax.numpy as jnp
from jax import lax
from jax.experimental import pallas as pl
from jax.experimental.pallas import tpu as pltpu
from jax.experimental.pallas import tpu_sc as plsc

D = 64
SCALE = 8.0
B = 4096 * 200
NC = 2
NS = 16
NW = NC * NS
BPW = B // NW
C = 128
NCHUNK = BPW // C
NBUF = 4


def _sc_gather(idx3, table):
    mesh = plsc.VectorSubcoreMesh(core_axis_name="c", subcore_axis_name="s")

    scratch = [pltpu.VMEM((NCHUNK, C), jnp.int32)]
    scratch += [pltpu.VMEM((C, D), jnp.float32) for _ in range(2 * NBUF)]
    scratch += [pltpu.SemaphoreType.DMA for _ in range(2 * NBUF)]

    @functools.partial(
        pl.kernel,
        mesh=mesh,
        out_type=jax.ShapeDtypeStruct((B, D), jnp.float32),
        scratch_types=scratch,
        compiler_params=pltpu.CompilerParams(use_tc_tiling_on_sc=False),
    )
    def kern(idx_hbm, table_hbm, out_hbm, idx_v, *bufs_and_sems):
        gbuf = bufs_and_sems[:NBUF]
        sbuf = bufs_and_sems[NBUF:2 * NBUF]
        gsem = bufs_and_sems[2 * NBUF:3 * NBUF]
        ssem = bufs_and_sems[3 * NBUF:4 * NBUF]

        wid = lax.axis_index("s") * NC + lax.axis_index("c")
        base = wid * BPW
        pltpu.sync_copy(idx_hbm.at[wid], idx_v)

        def gather(ci, b):
            pltpu.async_copy(table_hbm.at[idx_v.at[ci]], gbuf[b], gsem[b])

        def store(ci, b):
            pltpu.async_copy(
                sbuf[b], out_hbm.at[pl.ds(base + ci * C, C)], ssem[b])

        for b in range(NBUF):
            gather(b, b)

        def outer(i, _):
            cg = i * NBUF
            for b in range(NBUF):
                ci = cg + b
                pltpu.make_async_copy(
                    table_hbm.at[idx_v.at[ci]], gbuf[b], gsem[b]).wait()
                @pl.when(ci >= NBUF)
                def _():
                    pltpu.make_async_copy(
                        sbuf[b], out_hbm.at[pl.ds(base, C)], ssem[b]).wait()

                def srow(r, _):
                    for c in range(D // 16):
                        sl = pl.ds(c * 16, 16)
                        sbuf[b][r, sl] = gbuf[b][r, sl] * SCALE
                    return 0

                lax.fori_loop(0, C, srow, 0, unroll=2)
                store(ci, b)

                @pl.when(ci + NBUF < NCHUNK)
                def _():
                    gather(ci + NBUF, b)
            return 0

        lax.fori_loop(0, NCHUNK // NBUF, outer, 0)

        for b in range(NBUF):
            pltpu.make_async_copy(
                sbuf[b], out_hbm.at[pl.ds(base, C)], ssem[b]).wait()

    return kern(idx3, table)


def kernel(inp, emb_weight):
    idx3 = inp.reshape(NW, NCHUNK, C)
    out = _sc_gather(idx3, emb_weight)
    return out.reshape(4096, 200, D)
